# baseline (speedup 1.0000x reference)
import functools

import jax
import jax.numpy as jnp
from jax import lax
from jax.experimental import pallas as pl
from jax.experimental.pallas import tpu as pltpu
from jax.experimental.pallas import tpu_sc as plsc

N = 2_000_000
INV_N_CHARGES = 1.0 / 20.0
NW = 32
L = 16
BLK = 4000
NBLK = N // BLK
BASE = NBLK // NW
EXTRA = NBLK % NW
VPB = BLK // L

_mesh = plsc.VectorSubcoreMesh(core_axis_name="c", subcore_axis_name="s")


def _tile_range(w):
    nblk = jnp.where(w < EXTRA, BASE + 1, BASE)
    bstart = jnp.where(w < EXTRA, w * (BASE + 1),
                       EXTRA * (BASE + 1) + (w - EXTRA) * BASE)
    return bstart, nblk


@functools.partial(
    pl.kernel,
    out_type=jax.ShapeDtypeStruct((NW, 3, L), jnp.float32),
    mesh=_mesh,
    scratch_types=[
        pltpu.VMEM((BLK,), jnp.float32), pltpu.VMEM((BLK,), jnp.float32),
        pltpu.VMEM((BLK,), jnp.float32), pltpu.VMEM((BLK,), jnp.float32),
        pltpu.VMEM((BLK,), jnp.int32), pltpu.VMEM((BLK,), jnp.int32),
        pltpu.VMEM((3, L), jnp.float32),
        pltpu.SemaphoreType.DMA, pltpu.SemaphoreType.DMA,
        pltpu.SemaphoreType.DMA, pltpu.SemaphoreType.DMA,
        pltpu.SemaphoreType.DMA, pltpu.SemaphoreType.DMA,
    ],
)
def _reduce(x_hbm, c_hbm, i_hbm, out_hbm,
            xb0, xb1, cb0, cb1, ib0, ib1, pbuf,
            sx0, sx1, sc0, sc1, si0, si1):
    w = lax.axis_index("s") * 2 + lax.axis_index("c")
    bstart, nblk = _tile_range(w)
    zero = jnp.zeros((L,), jnp.float32)
    slots = ((xb0, cb0, ib0, sx0, sc0, si0),
             (xb1, cb1, ib1, sx1, sc1, si1))

    def issue(b, s):
        xb, cb, ib, sx, sc, si = slots[s]
        base = pl.multiple_of((bstart + b) * BLK, BLK)
        pltpu.async_copy(x_hbm.at[pl.ds(base, BLK)], xb, sx)
        pltpu.async_copy(c_hbm.at[pl.ds(base, BLK)], cb, sc)
        pltpu.async_copy(i_hbm.at[pl.ds(base, BLK)], ib, si)

    def wait_in(s):
        xb, cb, ib, sx, sc, si = slots[s]
        src = x_hbm.at[pl.ds(0, BLK)]
        pltpu.make_async_copy(src, xb, sx).wait()
        pltpu.make_async_copy(src, cb, sc).wait()
        isrc = i_hbm.at[pl.ds(0, BLK)]
        pltpu.make_async_copy(isrc, ib, si).wait()

    def acc_block(s):
        xb, cb, ib = slots[s][:3]
        carry = (pbuf[0, :], pbuf[1, :], pbuf[2, :])

        def vec_body(i, carry2):
            tot, s1, cnt = carry2
            xv = xb[pl.ds(i, L)]
            cv = cb[pl.ds(i, L)]
            iv = ib[pl.ds(i, L)]
            p = xv * cv
            ivf = iv.astype(jnp.float32)
            return tot + p, s1 + p * ivf, cnt + ivf

        tot, s1, cnt = plsc.parallel_loop(0, BLK, L, unroll=4,
                                          carry=carry)(vec_body)
        pbuf[0, :] = tot
        pbuf[1, :] = s1
        pbuf[2, :] = cnt

    pbuf[0, :] = zero
    pbuf[1, :] = zero
    pbuf[2, :] = zero
    issue(0, 0)
    issue(1, 1)

    def pair_body(pr, _):
        b = 2 * pr
        wait_in(0)
        acc_block(0)

        @pl.when(b + 2 < nblk)
        def _():
            issue(b + 2, 0)

        wait_in(1)
        acc_block(1)

        @pl.when(b + 3 < nblk)
        def _():
            issue(b + 3, 1)

        return 0

    lax.fori_loop(0, nblk // 2, pair_body, 0)

    @pl.when(lax.rem(nblk, 2) == 1)
    def _():
        wait_in(0)
        acc_block(0)

    pltpu.sync_copy(pbuf, out_hbm.at[w])


@functools.partial(
    pl.kernel,
    out_type=jax.ShapeDtypeStruct((N,), jnp.float32),
    mesh=_mesh,
    scratch_types=[
        pltpu.VMEM((BLK,), jnp.float32), pltpu.VMEM((BLK,), jnp.float32),
        pltpu.VMEM((BLK,), jnp.float32), pltpu.VMEM((BLK,), jnp.float32),
        pltpu.VMEM((BLK,), jnp.float32), pltpu.VMEM((BLK,), jnp.float32),
        pltpu.VMEM((NW, 3, L), jnp.float32),
        pltpu.VMEM((2 * L,), jnp.float32),
        pltpu.SemaphoreType.DMA, pltpu.SemaphoreType.DMA,
        pltpu.SemaphoreType.DMA, pltpu.SemaphoreType.DMA,
        pltpu.SemaphoreType.DMA, pltpu.SemaphoreType.DMA,
    ],
)
def _apply(x_hbm, c_hbm, p_hbm, out_hbm,
           xb0, xb1, cb0, cb1, ob0, ob1, pbuf, rbuf,
           sx0, sx1, sc0, sc1, so0, so1):
    w = lax.axis_index("s") * 2 + lax.axis_index("c")
    bstart, nblk = _tile_range(w)
    pltpu.sync_copy(p_hbm, pbuf)

    def all_lanes_sum(v):
        for k in (8, 4, 2, 1):
            rbuf[pl.ds(0, L)] = v
            rbuf[pl.ds(L, L)] = v
            v = v + rbuf[pl.ds(k, L)]
        return v

    totv = jnp.zeros((L,), jnp.float32)
    s1v = jnp.zeros((L,), jnp.float32)
    cntv = jnp.zeros((L,), jnp.float32)
    for t in range(NW):
        totv = totv + pbuf[t, 0, :]
        s1v = s1v + pbuf[t, 1, :]
        cntv = cntv + pbuf[t, 2, :]
    s1_all = all_lanes_sum(s1v)
    s0 = (all_lanes_sum(totv) - s1_all) * INV_N_CHARGES
    s1 = s1_all * INV_N_CHARGES
    bndf = jnp.float32(N) - all_lanes_sum(cntv)
    bnd = bndf.astype(jnp.int32)
    bnd_s = bndf[0].astype(jnp.int32)
    iota = lax.iota(jnp.int32, L)

    slots = ((xb0, cb0, ob0, sx0, sc0, so0),
             (xb1, cb1, ob1, sx1, sc1, so1))

    def issue(b, s):
        xb, cb = slots[s][:2]
        sx, sc = slots[s][3:5]
        base = pl.multiple_of((bstart + b) * BLK, BLK)
        pltpu.async_copy(x_hbm.at[pl.ds(base, BLK)], xb, sx)
        pltpu.async_copy(c_hbm.at[pl.ds(base, BLK)], cb, sc)

    def wait_in(s):
        xb, cb = slots[s][:2]
        sx, sc = slots[s][3:5]
        src = x_hbm.at[pl.ds(0, BLK)]
        pltpu.make_async_copy(src, xb, sx).wait()
        pltpu.make_async_copy(src, cb, sc).wait()

    def wait_scatter(s):
        ob, so = slots[s][2], slots[s][5]
        pltpu.make_async_copy(ob, out_hbm.at[pl.ds(0, BLK)], so).wait()

    def process(b, s):
        xb, cb, ob = slots[s][:3]
        so = slots[s][5]
        base = pl.multiple_of((bstart + b) * BLK, BLK)

        def const_body(segv):
            def vec_body(i):
                xv = xb[pl.ds(i, L)]
                cv = cb[pl.ds(i, L)]
                ob[pl.ds(i, L)] = (xv - segv) * cv
            return vec_body

        def sel_body(i):
            xv = xb[pl.ds(i, L)]
            cv = cb[pl.ds(i, L)]
            pos = iota + (base + i)
            sel = jnp.where(pos < bnd, s0, s1)
            ob[pl.ds(i, L)] = (xv - sel) * cv

        below = base + BLK <= bnd_s
        above = base >= bnd_s

        @pl.when(below)
        def _():
            plsc.parallel_loop(0, BLK, L, unroll=4)(const_body(s0))

        @pl.when(above)
        def _():
            plsc.parallel_loop(0, BLK, L, unroll=4)(const_body(s1))

        @pl.when(jnp.logical_not(jnp.logical_or(below, above)))
        def _():
            plsc.parallel_loop(0, BLK, L, unroll=4)(sel_body)

        pltpu.async_copy(ob, out_hbm.at[pl.ds(base, BLK)], so)

    issue(0, 0)
    issue(1, 1)

    def pair_body(pr, _):
        b = 2 * pr
        wait_in(0)

        @pl.when(pr > 0)
        def _():
            wait_scatter(0)

        process(b, 0)

        @pl.when(b + 2 < nblk)
        def _():
            issue(b + 2, 0)

        wait_in(1)

        @pl.when(pr > 0)
        def _():
            wait_scatter(1)

        process(b + 1, 1)

        @pl.when(b + 3 < nblk)
        def _():
            issue(b + 3, 1)

        return 0

    lax.fori_loop(0, nblk // 2, pair_body, 0)

    @pl.when(lax.rem(nblk, 2) == 1)
    def _():
        wait_in(0)
        wait_scatter(0)
        process(nblk - 1, 0)

    wait_scatter(0)
    wait_scatter(1)


def kernel(x0, charges, chg_idx):
    idx32 = chg_idx.astype(jnp.int32)
    partials = _reduce(x0, charges, idx32)
    return _apply(x0, charges, partials)


# speedup vs baseline: 1.0869x; 1.0869x over previous
import functools

import jax
import jax.numpy as jnp
from jax import lax
from jax.experimental import pallas as pl
from jax.experimental.pallas import tpu as pltpu
from jax.experimental.pallas import tpu_sc as plsc

N = 2_000_000
INV_N_CHARGES = 1.0 / 20.0
NW = 32
L = 16
BLK = 4000
NBLK = N // BLK
BASE = NBLK // NW
EXTRA = NBLK % NW
VPB = BLK // L

_mesh = plsc.VectorSubcoreMesh(core_axis_name="c", subcore_axis_name="s")


def _tile_range(w):
    nblk = jnp.where(w < EXTRA, BASE + 1, BASE)
    bstart = jnp.where(w < EXTRA, w * (BASE + 1),
                       EXTRA * (BASE + 1) + (w - EXTRA) * BASE)
    return bstart, nblk


@functools.partial(
    pl.kernel,
    out_type=jax.ShapeDtypeStruct((NW, 3, L), jnp.float32),
    mesh=_mesh,
    scratch_types=[
        pltpu.VMEM((BLK,), jnp.float32), pltpu.VMEM((BLK,), jnp.float32),
        pltpu.VMEM((BLK,), jnp.float32), pltpu.VMEM((BLK,), jnp.float32),
        pltpu.VMEM((BLK,), jnp.int32), pltpu.VMEM((BLK,), jnp.int32),
        pltpu.VMEM((3, L), jnp.float32),
        pltpu.SemaphoreType.DMA, pltpu.SemaphoreType.DMA,
        pltpu.SemaphoreType.DMA, pltpu.SemaphoreType.DMA,
        pltpu.SemaphoreType.DMA, pltpu.SemaphoreType.DMA,
    ],
)
def _reduce(x_hbm, c_hbm, i_hbm, out_hbm,
            xb0, xb1, cb0, cb1, ib0, ib1, pbuf,
            sx0, sx1, sc0, sc1, si0, si1):
    w = lax.axis_index("s") * 2 + lax.axis_index("c")
    bstart, nblk = _tile_range(w)
    zero = jnp.zeros((L,), jnp.float32)
    slots = ((xb0, cb0, ib0, sx0, sc0, si0),
             (xb1, cb1, ib1, sx1, sc1, si1))

    def issue(b, s):
        xb, cb, ib, sx, sc, si = slots[s]
        base = pl.multiple_of((bstart + b) * BLK, BLK)
        pltpu.async_copy(x_hbm.at[pl.ds(base, BLK)], xb, sx)
        pltpu.async_copy(c_hbm.at[pl.ds(base, BLK)], cb, sc)
        pltpu.async_copy(i_hbm.at[pl.ds(base, BLK)], ib, si)

    def wait_in(s):
        xb, cb, ib, sx, sc, si = slots[s]
        src = x_hbm.at[pl.ds(0, BLK)]
        pltpu.make_async_copy(src, xb, sx).wait()
        pltpu.make_async_copy(src, cb, sc).wait()
        isrc = i_hbm.at[pl.ds(0, BLK)]
        pltpu.make_async_copy(isrc, ib, si).wait()

    def acc_block(s):
        xb, cb, ib = slots[s][:3]
        zero = jnp.zeros((L,), jnp.float32)
        carry = (pbuf[0, :], pbuf[1, :], pbuf[2, :], zero, zero, zero)

        def vec_body(i, carry2):
            tot_a, s1_a, cnt_a, tot_b, s1_b, cnt_b = carry2
            xa = xb[pl.ds(i, L)]
            ca = cb[pl.ds(i, L)]
            ia = ib[pl.ds(i, L)]
            pa = xa * ca
            fa = ia.astype(jnp.float32)
            xbv = xb[pl.ds(i + L, L)]
            cbv = cb[pl.ds(i + L, L)]
            ibv = ib[pl.ds(i + L, L)]
            pb = xbv * cbv
            fb = ibv.astype(jnp.float32)
            return (tot_a + pa, s1_a + pa * fa, cnt_a + fa,
                    tot_b + pb, s1_b + pb * fb, cnt_b + fb)

        r = plsc.parallel_loop(0, BLK, 2 * L, unroll=2,
                               carry=carry)(vec_body)
        pbuf[0, :] = r[0] + r[3]
        pbuf[1, :] = r[1] + r[4]
        pbuf[2, :] = r[2] + r[5]

    pbuf[0, :] = zero
    pbuf[1, :] = zero
    pbuf[2, :] = zero
    issue(0, 0)
    issue(1, 1)

    def pair_body(pr, _):
        b = 2 * pr
        wait_in(0)
        acc_block(0)

        @pl.when(b + 2 < nblk)
        def _():
            issue(b + 2, 0)

        wait_in(1)
        acc_block(1)

        @pl.when(b + 3 < nblk)
        def _():
            issue(b + 3, 1)

        return 0

    lax.fori_loop(0, nblk // 2, pair_body, 0)

    @pl.when(lax.rem(nblk, 2) == 1)
    def _():
        wait_in(0)
        acc_block(0)

    pltpu.sync_copy(pbuf, out_hbm.at[w])


@functools.partial(
    pl.kernel,
    out_type=jax.ShapeDtypeStruct((N,), jnp.float32),
    mesh=_mesh,
    scratch_types=[
        pltpu.VMEM((BLK,), jnp.float32), pltpu.VMEM((BLK,), jnp.float32),
        pltpu.VMEM((BLK,), jnp.float32), pltpu.VMEM((BLK,), jnp.float32),
        pltpu.VMEM((BLK,), jnp.float32), pltpu.VMEM((BLK,), jnp.float32),
        pltpu.VMEM((NW, 3, L), jnp.float32),
        pltpu.VMEM((2 * L,), jnp.float32),
        pltpu.SemaphoreType.DMA, pltpu.SemaphoreType.DMA,
        pltpu.SemaphoreType.DMA, pltpu.SemaphoreType.DMA,
        pltpu.SemaphoreType.DMA, pltpu.SemaphoreType.DMA,
    ],
)
def _apply(x_hbm, c_hbm, p_hbm, out_hbm,
           xb0, xb1, cb0, cb1, ob0, ob1, pbuf, rbuf,
           sx0, sx1, sc0, sc1, so0, so1):
    w = lax.axis_index("s") * 2 + lax.axis_index("c")
    bstart, nblk = _tile_range(w)
    pltpu.sync_copy(p_hbm, pbuf)

    def all_lanes_sum(v):
        for k in (8, 4, 2, 1):
            rbuf[pl.ds(0, L)] = v
            rbuf[pl.ds(L, L)] = v
            v = v + rbuf[pl.ds(k, L)]
        return v

    totv = jnp.zeros((L,), jnp.float32)
    s1v = jnp.zeros((L,), jnp.float32)
    cntv = jnp.zeros((L,), jnp.float32)
    for t in range(NW):
        totv = totv + pbuf[t, 0, :]
        s1v = s1v + pbuf[t, 1, :]
        cntv = cntv + pbuf[t, 2, :]
    s1_all = all_lanes_sum(s1v)
    s0 = (all_lanes_sum(totv) - s1_all) * INV_N_CHARGES
    s1 = s1_all * INV_N_CHARGES
    bndf = jnp.float32(N) - all_lanes_sum(cntv)
    bnd = bndf.astype(jnp.int32)
    bnd_s = bndf[0].astype(jnp.int32)
    iota = lax.iota(jnp.int32, L)

    slots = ((xb0, cb0, ob0, sx0, sc0, so0),
             (xb1, cb1, ob1, sx1, sc1, so1))

    def issue(b, s):
        xb, cb = slots[s][:2]
        sx, sc = slots[s][3:5]
        base = pl.multiple_of((bstart + b) * BLK, BLK)
        pltpu.async_copy(x_hbm.at[pl.ds(base, BLK)], xb, sx)
        pltpu.async_copy(c_hbm.at[pl.ds(base, BLK)], cb, sc)

    def wait_in(s):
        xb, cb = slots[s][:2]
        sx, sc = slots[s][3:5]
        src = x_hbm.at[pl.ds(0, BLK)]
        pltpu.make_async_copy(src, xb, sx).wait()
        pltpu.make_async_copy(src, cb, sc).wait()

    def wait_scatter(s):
        ob, so = slots[s][2], slots[s][5]
        pltpu.make_async_copy(ob, out_hbm.at[pl.ds(0, BLK)], so).wait()

    def process(b, s):
        xb, cb, ob = slots[s][:3]
        so = slots[s][5]
        base = pl.multiple_of((bstart + b) * BLK, BLK)

        def const_body(segv):
            def vec_body(i):
                xv = xb[pl.ds(i, L)]
                cv = cb[pl.ds(i, L)]
                ob[pl.ds(i, L)] = (xv - segv) * cv
            return vec_body

        def sel_body(i):
            xv = xb[pl.ds(i, L)]
            cv = cb[pl.ds(i, L)]
            pos = iota + (base + i)
            sel = jnp.where(pos < bnd, s0, s1)
            ob[pl.ds(i, L)] = (xv - sel) * cv

        below = base + BLK <= bnd_s
        above = base >= bnd_s

        @pl.when(below)
        def _():
            plsc.parallel_loop(0, BLK, L, unroll=4)(const_body(s0))

        @pl.when(above)
        def _():
            plsc.parallel_loop(0, BLK, L, unroll=4)(const_body(s1))

        @pl.when(jnp.logical_not(jnp.logical_or(below, above)))
        def _():
            plsc.parallel_loop(0, BLK, L, unroll=4)(sel_body)

        pltpu.async_copy(ob, out_hbm.at[pl.ds(base, BLK)], so)

    issue(0, 0)
    issue(1, 1)

    def pair_body(pr, _):
        b = 2 * pr
        wait_in(0)

        @pl.when(pr > 0)
        def _():
            wait_scatter(0)

        process(b, 0)

        @pl.when(b + 2 < nblk)
        def _():
            issue(b + 2, 0)

        wait_in(1)

        @pl.when(pr > 0)
        def _():
            wait_scatter(1)

        process(b + 1, 1)

        @pl.when(b + 3 < nblk)
        def _():
            issue(b + 3, 1)

        return 0

    lax.fori_loop(0, nblk // 2, pair_body, 0)

    @pl.when(lax.rem(nblk, 2) == 1)
    def _():
        wait_in(0)
        wait_scatter(0)
        process(nblk - 1, 0)

    wait_scatter(0)
    wait_scatter(1)


def kernel(x0, charges, chg_idx):
    idx32 = chg_idx.astype(jnp.int32)
    partials = _reduce(x0, charges, idx32)
    return _apply(x0, charges, partials)


# speedup vs baseline: 1.1646x; 1.0715x over previous
import functools

import jax
import jax.numpy as jnp
from jax import lax
from jax.experimental import pallas as pl
from jax.experimental.pallas import tpu as pltpu
from jax.experimental.pallas import tpu_sc as plsc

N = 2_000_000
INV_N_CHARGES = 1.0 / 20.0
NW = 32
L = 16
BLK = 8000
NBLK = N // BLK
BASE = NBLK // NW
EXTRA = NBLK % NW
VPB = BLK // L

_mesh = plsc.VectorSubcoreMesh(core_axis_name="c", subcore_axis_name="s")


def _tile_range(w):
    nblk = jnp.where(w < EXTRA, BASE + 1, BASE)
    bstart = jnp.where(w < EXTRA, w * (BASE + 1),
                       EXTRA * (BASE + 1) + (w - EXTRA) * BASE)
    return bstart, nblk


@functools.partial(
    pl.kernel,
    out_type=jax.ShapeDtypeStruct((NW, 3, L), jnp.float32),
    mesh=_mesh,
    scratch_types=[
        pltpu.VMEM((BLK,), jnp.float32), pltpu.VMEM((BLK,), jnp.float32),
        pltpu.VMEM((BLK,), jnp.float32), pltpu.VMEM((BLK,), jnp.float32),
        pltpu.VMEM((BLK,), jnp.int32), pltpu.VMEM((BLK,), jnp.int32),
        pltpu.VMEM((3, L), jnp.float32),
        pltpu.SemaphoreType.DMA, pltpu.SemaphoreType.DMA,
        pltpu.SemaphoreType.DMA, pltpu.SemaphoreType.DMA,
        pltpu.SemaphoreType.DMA, pltpu.SemaphoreType.DMA,
    ],
)
def _reduce(x_hbm, c_hbm, i_hbm, out_hbm,
            xb0, xb1, cb0, cb1, ib0, ib1, pbuf,
            sx0, sx1, sc0, sc1, si0, si1):
    w = lax.axis_index("s") * 2 + lax.axis_index("c")
    bstart, nblk = _tile_range(w)
    zero = jnp.zeros((L,), jnp.float32)
    slots = ((xb0, cb0, ib0, sx0, sc0, si0),
             (xb1, cb1, ib1, sx1, sc1, si1))

    def issue(b, s):
        xb, cb, ib, sx, sc, si = slots[s]
        base = pl.multiple_of((bstart + b) * BLK, BLK)
        pltpu.async_copy(x_hbm.at[pl.ds(base, BLK)], xb, sx)
        pltpu.async_copy(c_hbm.at[pl.ds(base, BLK)], cb, sc)
        pltpu.async_copy(i_hbm.at[pl.ds(base, BLK)], ib, si)

    def wait_in(s):
        xb, cb, ib, sx, sc, si = slots[s]
        src = x_hbm.at[pl.ds(0, BLK)]
        pltpu.make_async_copy(src, xb, sx).wait()
        pltpu.make_async_copy(src, cb, sc).wait()
        isrc = i_hbm.at[pl.ds(0, BLK)]
        pltpu.make_async_copy(isrc, ib, si).wait()

    def acc_block(s):
        xb, cb, ib = slots[s][:3]
        zero = jnp.zeros((L,), jnp.float32)
        carry = (pbuf[0, :], pbuf[1, :], pbuf[2, :], zero, zero, zero)

        def vec_body(i, carry2):
            tot_a, s1_a, cnt_a, tot_b, s1_b, cnt_b = carry2
            xa = xb[pl.ds(i, L)]
            ca = cb[pl.ds(i, L)]
            ia = ib[pl.ds(i, L)]
            pa = xa * ca
            fa = ia.astype(jnp.float32)
            xbv = xb[pl.ds(i + L, L)]
            cbv = cb[pl.ds(i + L, L)]
            ibv = ib[pl.ds(i + L, L)]
            pb = xbv * cbv
            fb = ibv.astype(jnp.float32)
            return (tot_a + pa, s1_a + pa * fa, cnt_a + fa,
                    tot_b + pb, s1_b + pb * fb, cnt_b + fb)

        r = plsc.parallel_loop(0, BLK, 2 * L, unroll=2,
                               carry=carry)(vec_body)
        pbuf[0, :] = r[0] + r[3]
        pbuf[1, :] = r[1] + r[4]
        pbuf[2, :] = r[2] + r[5]

    pbuf[0, :] = zero
    pbuf[1, :] = zero
    pbuf[2, :] = zero
    issue(0, 0)
    issue(1, 1)

    def pair_body(pr, _):
        b = 2 * pr
        wait_in(0)
        acc_block(0)

        @pl.when(b + 2 < nblk)
        def _():
            issue(b + 2, 0)

        wait_in(1)
        acc_block(1)

        @pl.when(b + 3 < nblk)
        def _():
            issue(b + 3, 1)

        return 0

    lax.fori_loop(0, nblk // 2, pair_body, 0)

    @pl.when(lax.rem(nblk, 2) == 1)
    def _():
        wait_in(0)
        acc_block(0)

    pltpu.sync_copy(pbuf, out_hbm.at[w])


@functools.partial(
    pl.kernel,
    out_type=jax.ShapeDtypeStruct((N,), jnp.float32),
    mesh=_mesh,
    scratch_types=[
        pltpu.VMEM((BLK,), jnp.float32), pltpu.VMEM((BLK,), jnp.float32),
        pltpu.VMEM((BLK,), jnp.float32), pltpu.VMEM((BLK,), jnp.float32),
        pltpu.VMEM((BLK,), jnp.float32), pltpu.VMEM((BLK,), jnp.float32),
        pltpu.VMEM((NW, 3, L), jnp.float32),
        pltpu.VMEM((2 * L,), jnp.float32),
        pltpu.SemaphoreType.DMA, pltpu.SemaphoreType.DMA,
        pltpu.SemaphoreType.DMA, pltpu.SemaphoreType.DMA,
        pltpu.SemaphoreType.DMA, pltpu.SemaphoreType.DMA,
    ],
)
def _apply(x_hbm, c_hbm, p_hbm, out_hbm,
           xb0, xb1, cb0, cb1, ob0, ob1, pbuf, rbuf,
           sx0, sx1, sc0, sc1, so0, so1):
    w = lax.axis_index("s") * 2 + lax.axis_index("c")
    bstart, nblk = _tile_range(w)
    pltpu.sync_copy(p_hbm, pbuf)

    def all_lanes_sum(v):
        for k in (8, 4, 2, 1):
            rbuf[pl.ds(0, L)] = v
            rbuf[pl.ds(L, L)] = v
            v = v + rbuf[pl.ds(k, L)]
        return v

    totv = jnp.zeros((L,), jnp.float32)
    s1v = jnp.zeros((L,), jnp.float32)
    cntv = jnp.zeros((L,), jnp.float32)
    for t in range(NW):
        totv = totv + pbuf[t, 0, :]
        s1v = s1v + pbuf[t, 1, :]
        cntv = cntv + pbuf[t, 2, :]
    s1_all = all_lanes_sum(s1v)
    s0 = (all_lanes_sum(totv) - s1_all) * INV_N_CHARGES
    s1 = s1_all * INV_N_CHARGES
    bndf = jnp.float32(N) - all_lanes_sum(cntv)
    bnd = bndf.astype(jnp.int32)
    bnd_s = bndf[0].astype(jnp.int32)
    iota = lax.iota(jnp.int32, L)

    slots = ((xb0, cb0, ob0, sx0, sc0, so0),
             (xb1, cb1, ob1, sx1, sc1, so1))

    def issue(b, s):
        xb, cb = slots[s][:2]
        sx, sc = slots[s][3:5]
        base = pl.multiple_of((bstart + b) * BLK, BLK)
        pltpu.async_copy(x_hbm.at[pl.ds(base, BLK)], xb, sx)
        pltpu.async_copy(c_hbm.at[pl.ds(base, BLK)], cb, sc)

    def wait_in(s):
        xb, cb = slots[s][:2]
        sx, sc = slots[s][3:5]
        src = x_hbm.at[pl.ds(0, BLK)]
        pltpu.make_async_copy(src, xb, sx).wait()
        pltpu.make_async_copy(src, cb, sc).wait()

    def wait_scatter(s):
        ob, so = slots[s][2], slots[s][5]
        pltpu.make_async_copy(ob, out_hbm.at[pl.ds(0, BLK)], so).wait()

    def process(b, s):
        xb, cb, ob = slots[s][:3]
        so = slots[s][5]
        base = pl.multiple_of((bstart + b) * BLK, BLK)

        def const_body(segv):
            def vec_body(i):
                xv = xb[pl.ds(i, L)]
                cv = cb[pl.ds(i, L)]
                ob[pl.ds(i, L)] = (xv - segv) * cv
            return vec_body

        def sel_body(i):
            xv = xb[pl.ds(i, L)]
            cv = cb[pl.ds(i, L)]
            pos = iota + (base + i)
            sel = jnp.where(pos < bnd, s0, s1)
            ob[pl.ds(i, L)] = (xv - sel) * cv

        below = base + BLK <= bnd_s
        above = base >= bnd_s

        @pl.when(below)
        def _():
            plsc.parallel_loop(0, BLK, L, unroll=4)(const_body(s0))

        @pl.when(above)
        def _():
            plsc.parallel_loop(0, BLK, L, unroll=4)(const_body(s1))

        @pl.when(jnp.logical_not(jnp.logical_or(below, above)))
        def _():
            plsc.parallel_loop(0, BLK, L, unroll=4)(sel_body)

        pltpu.async_copy(ob, out_hbm.at[pl.ds(base, BLK)], so)

    issue(0, 0)
    issue(1, 1)

    def pair_body(pr, _):
        b = 2 * pr
        wait_in(0)

        @pl.when(pr > 0)
        def _():
            wait_scatter(0)

        process(b, 0)

        @pl.when(b + 2 < nblk)
        def _():
            issue(b + 2, 0)

        wait_in(1)

        @pl.when(pr > 0)
        def _():
            wait_scatter(1)

        process(b + 1, 1)

        @pl.when(b + 3 < nblk)
        def _():
            issue(b + 3, 1)

        return 0

    lax.fori_loop(0, nblk // 2, pair_body, 0)

    @pl.when(lax.rem(nblk, 2) == 1)
    def _():
        wait_in(0)
        wait_scatter(0)
        process(nblk - 1, 0)

    wait_scatter(0)
    wait_scatter(1)


def kernel(x0, charges, chg_idx):
    idx32 = chg_idx.astype(jnp.int32)
    partials = _reduce(x0, charges, idx32)
    return _apply(x0, charges, partials)


# speedup vs baseline: 1.1646x; 1.0000x over previous
import functools

import jax
import jax.numpy as jnp
from jax import lax
from jax.experimental import pallas as pl
from jax.experimental.pallas import tpu as pltpu
from jax.experimental.pallas import tpu_sc as plsc

N = 2_000_000
INV_N_CHARGES = 1.0 / 20.0
NW = 32
L = 16
BLK = 16000
NBLK = N // BLK
BASE = NBLK // NW
EXTRA = NBLK % NW
VPB = BLK // L

_mesh = plsc.VectorSubcoreMesh(core_axis_name="c", subcore_axis_name="s")


def _tile_range(w):
    nblk = jnp.where(w < EXTRA, BASE + 1, BASE)
    bstart = jnp.where(w < EXTRA, w * (BASE + 1),
                       EXTRA * (BASE + 1) + (w - EXTRA) * BASE)
    return bstart, nblk


@functools.partial(
    pl.kernel,
    out_type=jax.ShapeDtypeStruct((NW, 3, L), jnp.float32),
    mesh=_mesh,
    scratch_types=[
        pltpu.VMEM((BLK,), jnp.float32), pltpu.VMEM((BLK,), jnp.float32),
        pltpu.VMEM((BLK,), jnp.float32), pltpu.VMEM((BLK,), jnp.float32),
        pltpu.VMEM((BLK,), jnp.int32), pltpu.VMEM((BLK,), jnp.int32),
        pltpu.VMEM((3, L), jnp.float32),
        pltpu.SemaphoreType.DMA, pltpu.SemaphoreType.DMA,
        pltpu.SemaphoreType.DMA, pltpu.SemaphoreType.DMA,
        pltpu.SemaphoreType.DMA, pltpu.SemaphoreType.DMA,
    ],
)
def _reduce(x_hbm, c_hbm, i_hbm, out_hbm,
            xb0, xb1, cb0, cb1, ib0, ib1, pbuf,
            sx0, sx1, sc0, sc1, si0, si1):
    w = lax.axis_index("s") * 2 + lax.axis_index("c")
    bstart, nblk = _tile_range(w)
    zero = jnp.zeros((L,), jnp.float32)
    slots = ((xb0, cb0, ib0, sx0, sc0, si0),
             (xb1, cb1, ib1, sx1, sc1, si1))

    def issue(b, s):
        xb, cb, ib, sx, sc, si = slots[s]
        base = pl.multiple_of((bstart + b) * BLK, BLK)
        pltpu.async_copy(x_hbm.at[pl.ds(base, BLK)], xb, sx)
        pltpu.async_copy(c_hbm.at[pl.ds(base, BLK)], cb, sc)
        pltpu.async_copy(i_hbm.at[pl.ds(base, BLK)], ib, si)

    def wait_in(s):
        xb, cb, ib, sx, sc, si = slots[s]
        src = x_hbm.at[pl.ds(0, BLK)]
        pltpu.make_async_copy(src, xb, sx).wait()
        pltpu.make_async_copy(src, cb, sc).wait()
        isrc = i_hbm.at[pl.ds(0, BLK)]
        pltpu.make_async_copy(isrc, ib, si).wait()

    def acc_block(s):
        xb, cb, ib = slots[s][:3]
        zero = jnp.zeros((L,), jnp.float32)
        carry = (pbuf[0, :], pbuf[1, :], pbuf[2, :], zero, zero, zero)

        def vec_body(i, carry2):
            tot_a, s1_a, cnt_a, tot_b, s1_b, cnt_b = carry2
            xa = xb[pl.ds(i, L)]
            ca = cb[pl.ds(i, L)]
            ia = ib[pl.ds(i, L)]
            pa = xa * ca
            fa = ia.astype(jnp.float32)
            xbv = xb[pl.ds(i + L, L)]
            cbv = cb[pl.ds(i + L, L)]
            ibv = ib[pl.ds(i + L, L)]
            pb = xbv * cbv
            fb = ibv.astype(jnp.float32)
            return (tot_a + pa, s1_a + pa * fa, cnt_a + fa,
                    tot_b + pb, s1_b + pb * fb, cnt_b + fb)

        r = plsc.parallel_loop(0, BLK, 2 * L, unroll=2,
                               carry=carry)(vec_body)
        pbuf[0, :] = r[0] + r[3]
        pbuf[1, :] = r[1] + r[4]
        pbuf[2, :] = r[2] + r[5]

    pbuf[0, :] = zero
    pbuf[1, :] = zero
    pbuf[2, :] = zero
    issue(0, 0)
    issue(1, 1)

    def pair_body(pr, _):
        b = 2 * pr
        wait_in(0)
        acc_block(0)

        @pl.when(b + 2 < nblk)
        def _():
            issue(b + 2, 0)

        wait_in(1)
        acc_block(1)

        @pl.when(b + 3 < nblk)
        def _():
            issue(b + 3, 1)

        return 0

    lax.fori_loop(0, nblk // 2, pair_body, 0)

    @pl.when(lax.rem(nblk, 2) == 1)
    def _():
        wait_in(0)
        acc_block(0)

    pltpu.sync_copy(pbuf, out_hbm.at[w])


@functools.partial(
    pl.kernel,
    out_type=jax.ShapeDtypeStruct((N,), jnp.float32),
    mesh=_mesh,
    scratch_types=[
        pltpu.VMEM((BLK,), jnp.float32), pltpu.VMEM((BLK,), jnp.float32),
        pltpu.VMEM((BLK,), jnp.float32), pltpu.VMEM((BLK,), jnp.float32),
        pltpu.VMEM((BLK,), jnp.float32), pltpu.VMEM((BLK,), jnp.float32),
        pltpu.VMEM((NW, 3, L), jnp.float32),
        pltpu.VMEM((2 * L,), jnp.float32),
        pltpu.SemaphoreType.DMA, pltpu.SemaphoreType.DMA,
        pltpu.SemaphoreType.DMA, pltpu.SemaphoreType.DMA,
        pltpu.SemaphoreType.DMA, pltpu.SemaphoreType.DMA,
    ],
)
def _apply(x_hbm, c_hbm, p_hbm, out_hbm,
           xb0, xb1, cb0, cb1, ob0, ob1, pbuf, rbuf,
           sx0, sx1, sc0, sc1, so0, so1):
    w = lax.axis_index("s") * 2 + lax.axis_index("c")
    bstart, nblk = _tile_range(w)
    pltpu.sync_copy(p_hbm, pbuf)

    def all_lanes_sum(v):
        for k in (8, 4, 2, 1):
            rbuf[pl.ds(0, L)] = v
            rbuf[pl.ds(L, L)] = v
            v = v + rbuf[pl.ds(k, L)]
        return v

    totv = jnp.zeros((L,), jnp.float32)
    s1v = jnp.zeros((L,), jnp.float32)
    cntv = jnp.zeros((L,), jnp.float32)
    for t in range(NW):
        totv = totv + pbuf[t, 0, :]
        s1v = s1v + pbuf[t, 1, :]
        cntv = cntv + pbuf[t, 2, :]
    s1_all = all_lanes_sum(s1v)
    s0 = (all_lanes_sum(totv) - s1_all) * INV_N_CHARGES
    s1 = s1_all * INV_N_CHARGES
    bndf = jnp.float32(N) - all_lanes_sum(cntv)
    bnd = bndf.astype(jnp.int32)
    bnd_s = bndf[0].astype(jnp.int32)
    iota = lax.iota(jnp.int32, L)

    slots = ((xb0, cb0, ob0, sx0, sc0, so0),
             (xb1, cb1, ob1, sx1, sc1, so1))

    def issue(b, s):
        xb, cb = slots[s][:2]
        sx, sc = slots[s][3:5]
        base = pl.multiple_of((bstart + b) * BLK, BLK)
        pltpu.async_copy(x_hbm.at[pl.ds(base, BLK)], xb, sx)
        pltpu.async_copy(c_hbm.at[pl.ds(base, BLK)], cb, sc)

    def wait_in(s):
        xb, cb = slots[s][:2]
        sx, sc = slots[s][3:5]
        src = x_hbm.at[pl.ds(0, BLK)]
        pltpu.make_async_copy(src, xb, sx).wait()
        pltpu.make_async_copy(src, cb, sc).wait()

    def wait_scatter(s):
        ob, so = slots[s][2], slots[s][5]
        pltpu.make_async_copy(ob, out_hbm.at[pl.ds(0, BLK)], so).wait()

    def process(b, s):
        xb, cb, ob = slots[s][:3]
        so = slots[s][5]
        base = pl.multiple_of((bstart + b) * BLK, BLK)

        def const_body(segv):
            def vec_body(i):
                xv = xb[pl.ds(i, L)]
                cv = cb[pl.ds(i, L)]
                ob[pl.ds(i, L)] = (xv - segv) * cv
            return vec_body

        def sel_body(i):
            xv = xb[pl.ds(i, L)]
            cv = cb[pl.ds(i, L)]
            pos = iota + (base + i)
            sel = jnp.where(pos < bnd, s0, s1)
            ob[pl.ds(i, L)] = (xv - sel) * cv

        below = base + BLK <= bnd_s
        above = base >= bnd_s

        @pl.when(below)
        def _():
            plsc.parallel_loop(0, BLK, L, unroll=4)(const_body(s0))

        @pl.when(above)
        def _():
            plsc.parallel_loop(0, BLK, L, unroll=4)(const_body(s1))

        @pl.when(jnp.logical_not(jnp.logical_or(below, above)))
        def _():
            plsc.parallel_loop(0, BLK, L, unroll=4)(sel_body)

        pltpu.async_copy(ob, out_hbm.at[pl.ds(base, BLK)], so)

    issue(0, 0)
    issue(1, 1)

    def pair_body(pr, _):
        b = 2 * pr
        wait_in(0)

        @pl.when(pr > 0)
        def _():
            wait_scatter(0)

        process(b, 0)

        @pl.when(b + 2 < nblk)
        def _():
            issue(b + 2, 0)

        wait_in(1)

        @pl.when(pr > 0)
        def _():
            wait_scatter(1)

        process(b + 1, 1)

        @pl.when(b + 3 < nblk)
        def _():
            issue(b + 3, 1)

        return 0

    lax.fori_loop(0, nblk // 2, pair_body, 0)

    @pl.when(lax.rem(nblk, 2) == 1)
    def _():
        wait_in(0)
        wait_scatter(0)
        process(nblk - 1, 0)

    wait_scatter(0)
    wait_scatter(1)


def kernel(x0, charges, chg_idx):
    idx32 = chg_idx.astype(jnp.int32)
    partials = _reduce(x0, charges, idx32)
    return _apply(x0, charges, partials)


# speedup vs baseline: 1.1674x; 1.0024x over previous
import functools

import jax
import jax.numpy as jnp
from jax import lax
from jax.experimental import pallas as pl
from jax.experimental.pallas import tpu as pltpu
from jax.experimental.pallas import tpu_sc as plsc

N = 2_000_000
INV_N_CHARGES = 1.0 / 20.0
NW = 32
L = 16
BLK = 8000
NBLK = N // BLK
BASE = NBLK // NW
EXTRA = NBLK % NW
VPB = BLK // L

_mesh = plsc.VectorSubcoreMesh(core_axis_name="c", subcore_axis_name="s")


def _tile_range(w):
    nblk = jnp.where(w < EXTRA, BASE + 1, BASE)
    bstart = jnp.where(w < EXTRA, w * (BASE + 1),
                       EXTRA * (BASE + 1) + (w - EXTRA) * BASE)
    return bstart, nblk


@functools.partial(
    pl.kernel,
    out_type=jax.ShapeDtypeStruct((NW, 3, L), jnp.float32),
    mesh=_mesh,
    scratch_types=[
        pltpu.VMEM((BLK,), jnp.float32), pltpu.VMEM((BLK,), jnp.float32),
        pltpu.VMEM((BLK,), jnp.float32), pltpu.VMEM((BLK,), jnp.float32),
        pltpu.VMEM((BLK,), jnp.int32), pltpu.VMEM((BLK,), jnp.int32),
        pltpu.VMEM((3, L), jnp.float32),
        pltpu.SemaphoreType.DMA, pltpu.SemaphoreType.DMA,
        pltpu.SemaphoreType.DMA, pltpu.SemaphoreType.DMA,
        pltpu.SemaphoreType.DMA, pltpu.SemaphoreType.DMA,
    ],
)
def _reduce(x_hbm, c_hbm, i_hbm, out_hbm,
            xb0, xb1, cb0, cb1, ib0, ib1, pbuf,
            sx0, sx1, sc0, sc1, si0, si1):
    w = lax.axis_index("s") * 2 + lax.axis_index("c")
    bstart, nblk = _tile_range(w)
    zero = jnp.zeros((L,), jnp.float32)
    slots = ((xb0, cb0, ib0, sx0, sc0, si0),
             (xb1, cb1, ib1, sx1, sc1, si1))

    def issue(b, s):
        xb, cb, ib, sx, sc, si = slots[s]
        base = pl.multiple_of((bstart + b) * BLK, BLK)
        pltpu.async_copy(x_hbm.at[pl.ds(base, BLK)], xb, sx)
        pltpu.async_copy(c_hbm.at[pl.ds(base, BLK)], cb, sc)
        pltpu.async_copy(i_hbm.at[pl.ds(base, BLK)], ib, si)

    def wait_in(s):
        xb, cb, ib, sx, sc, si = slots[s]
        src = x_hbm.at[pl.ds(0, BLK)]
        pltpu.make_async_copy(src, xb, sx).wait()
        pltpu.make_async_copy(src, cb, sc).wait()
        isrc = i_hbm.at[pl.ds(0, BLK)]
        pltpu.make_async_copy(isrc, ib, si).wait()

    def acc_block(s):
        xb, cb, ib = slots[s][:3]
        zero = jnp.zeros((L,), jnp.float32)
        carry = (pbuf[0, :], pbuf[1, :], pbuf[2, :], zero, zero, zero)

        def vec_body(i, carry2):
            tot_a, s1_a, cnt_a, tot_b, s1_b, cnt_b = carry2
            xa = xb[pl.ds(i, L)]
            ca = cb[pl.ds(i, L)]
            ia = ib[pl.ds(i, L)]
            pa = xa * ca
            fa = ia.astype(jnp.float32)
            xbv = xb[pl.ds(i + L, L)]
            cbv = cb[pl.ds(i + L, L)]
            ibv = ib[pl.ds(i + L, L)]
            pb = xbv * cbv
            fb = ibv.astype(jnp.float32)
            return (tot_a + pa, s1_a + pa * fa, cnt_a + fa,
                    tot_b + pb, s1_b + pb * fb, cnt_b + fb)

        r = plsc.parallel_loop(0, BLK, 2 * L, unroll=2,
                               carry=carry)(vec_body)
        pbuf[0, :] = r[0] + r[3]
        pbuf[1, :] = r[1] + r[4]
        pbuf[2, :] = r[2] + r[5]

    pbuf[0, :] = zero
    pbuf[1, :] = zero
    pbuf[2, :] = zero
    issue(0, 0)
    issue(1, 1)

    def pair_body(pr, _):
        b = 2 * pr
        wait_in(0)
        acc_block(0)

        @pl.when(b + 2 < nblk)
        def _():
            issue(b + 2, 0)

        wait_in(1)
        acc_block(1)

        @pl.when(b + 3 < nblk)
        def _():
            issue(b + 3, 1)

        return 0

    lax.fori_loop(0, nblk // 2, pair_body, 0)

    @pl.when(lax.rem(nblk, 2) == 1)
    def _():
        wait_in(0)
        acc_block(0)

    pltpu.sync_copy(pbuf, out_hbm.at[w])


@functools.partial(
    pl.kernel,
    out_type=jax.ShapeDtypeStruct((N,), jnp.float32),
    mesh=_mesh,
    scratch_types=[
        pltpu.VMEM((BLK,), jnp.float32), pltpu.VMEM((BLK,), jnp.float32),
        pltpu.VMEM((BLK,), jnp.float32), pltpu.VMEM((BLK,), jnp.float32),
        pltpu.VMEM((BLK,), jnp.float32), pltpu.VMEM((BLK,), jnp.float32),
        pltpu.VMEM((NW, 3, L), jnp.float32),
        pltpu.VMEM((2 * L,), jnp.float32),
        pltpu.SemaphoreType.DMA, pltpu.SemaphoreType.DMA,
        pltpu.SemaphoreType.DMA, pltpu.SemaphoreType.DMA,
        pltpu.SemaphoreType.DMA, pltpu.SemaphoreType.DMA,
    ],
)
def _apply(x_hbm, c_hbm, p_hbm, out_hbm,
           xb0, xb1, cb0, cb1, ob0, ob1, pbuf, rbuf,
           sx0, sx1, sc0, sc1, so0, so1):
    w = lax.axis_index("s") * 2 + lax.axis_index("c")
    bstart, nblk = _tile_range(w)
    pltpu.sync_copy(p_hbm, pbuf)

    def all_lanes_sum(v):
        for k in (8, 4, 2, 1):
            rbuf[pl.ds(0, L)] = v
            rbuf[pl.ds(L, L)] = v
            v = v + rbuf[pl.ds(k, L)]
        return v

    totv = jnp.zeros((L,), jnp.float32)
    s1v = jnp.zeros((L,), jnp.float32)
    cntv = jnp.zeros((L,), jnp.float32)
    for t in range(NW):
        totv = totv + pbuf[t, 0, :]
        s1v = s1v + pbuf[t, 1, :]
        cntv = cntv + pbuf[t, 2, :]
    s1_all = all_lanes_sum(s1v)
    s0 = (all_lanes_sum(totv) - s1_all) * INV_N_CHARGES
    s1 = s1_all * INV_N_CHARGES
    bndf = jnp.float32(N) - all_lanes_sum(cntv)
    bnd = bndf.astype(jnp.int32)
    bnd_s = bndf[0].astype(jnp.int32)
    iota = lax.iota(jnp.int32, L)

    slots = ((xb0, cb0, ob0, sx0, sc0, so0),
             (xb1, cb1, ob1, sx1, sc1, so1))

    def issue(b, s):
        xb, cb = slots[s][:2]
        sx, sc = slots[s][3:5]
        base = pl.multiple_of((bstart + b) * BLK, BLK)
        pltpu.async_copy(x_hbm.at[pl.ds(base, BLK)], xb, sx)
        pltpu.async_copy(c_hbm.at[pl.ds(base, BLK)], cb, sc)

    def wait_in(s):
        xb, cb = slots[s][:2]
        sx, sc = slots[s][3:5]
        src = x_hbm.at[pl.ds(0, BLK)]
        pltpu.make_async_copy(src, xb, sx).wait()
        pltpu.make_async_copy(src, cb, sc).wait()

    def wait_scatter(s):
        ob, so = slots[s][2], slots[s][5]
        pltpu.make_async_copy(ob, out_hbm.at[pl.ds(0, BLK)], so).wait()

    def process(b, s):
        xb, cb, ob = slots[s][:3]
        so = slots[s][5]
        base = pl.multiple_of((bstart + b) * BLK, BLK)

        def const_body(segv):
            def vec_body(i):
                xv = xb[pl.ds(i, L)]
                cv = cb[pl.ds(i, L)]
                ob[pl.ds(i, L)] = (xv - segv) * cv
            return vec_body

        def sel_body(i):
            xv = xb[pl.ds(i, L)]
            cv = cb[pl.ds(i, L)]
            pos = iota + (base + i)
            sel = jnp.where(pos < bnd, s0, s1)
            ob[pl.ds(i, L)] = (xv - sel) * cv

        below = base + BLK <= bnd_s
        above = base >= bnd_s

        @pl.when(below)
        def _():
            plsc.parallel_loop(0, BLK, L, unroll=4)(const_body(s0))

        @pl.when(above)
        def _():
            plsc.parallel_loop(0, BLK, L, unroll=4)(const_body(s1))

        @pl.when(jnp.logical_not(jnp.logical_or(below, above)))
        def _():
            plsc.parallel_loop(0, BLK, L, unroll=4)(sel_body)

        pltpu.async_copy(ob, out_hbm.at[pl.ds(base, BLK)], so)

    issue(0, 0)
    issue(1, 1)

    def pair_body(pr, _):
        b = 2 * pr
        wait_in(0)

        @pl.when(pr > 0)
        def _():
            wait_scatter(0)

        process(b, 0)

        @pl.when(b + 2 < nblk)
        def _():
            issue(b + 2, 0)

        wait_in(1)

        @pl.when(pr > 0)
        def _():
            wait_scatter(1)

        process(b + 1, 1)

        @pl.when(b + 3 < nblk)
        def _():
            issue(b + 3, 1)

        return 0

    lax.fori_loop(0, nblk // 2, pair_body, 0)

    @pl.when(lax.rem(nblk, 2) == 1)
    def _():
        wait_in(0)
        wait_scatter(0)
        process(nblk - 1, 0)

    wait_scatter(0)
    wait_scatter(1)


def kernel(x0, charges, chg_idx):
    idx32 = chg_idx.astype(jnp.int32)
    partials = _reduce(x0, charges, idx32)
    return _apply(x0, charges, partials)


# speedup vs baseline: 1.2779x; 1.0946x over previous
import functools

import jax
import jax.numpy as jnp
from jax import lax
from jax.experimental import pallas as pl
from jax.experimental.pallas import tpu as pltpu
from jax.experimental.pallas import tpu_sc as plsc

N = 2_000_000
INV_N_CHARGES = 1.0 / 20.0
NW = 32
L = 16
BLK = 8000
NBLK = N // BLK
BASE = NBLK // NW
EXTRA = NBLK % NW
VPB = BLK // L

_mesh = plsc.VectorSubcoreMesh(core_axis_name="c", subcore_axis_name="s")


def _tile_range(w):
    nblk = jnp.where(w < EXTRA, BASE + 1, BASE)
    bstart = jnp.where(w < EXTRA, w * (BASE + 1),
                       EXTRA * (BASE + 1) + (w - EXTRA) * BASE)
    return bstart, nblk


@functools.partial(
    pl.kernel,
    out_type=(jax.ShapeDtypeStruct((N,), jnp.float32),
              jax.ShapeDtypeStruct((NW, 3, L), jnp.float32)),
    mesh=_mesh,
    scratch_types=[
        pltpu.VMEM((BLK,), jnp.float32), pltpu.VMEM((BLK,), jnp.float32),
        pltpu.VMEM((BLK,), jnp.float32), pltpu.VMEM((BLK,), jnp.float32),
        pltpu.VMEM((BLK,), jnp.int32), pltpu.VMEM((BLK,), jnp.int32),
        pltpu.VMEM((BLK,), jnp.float32), pltpu.VMEM((BLK,), jnp.float32),
        pltpu.VMEM((3, L), jnp.float32),
        pltpu.VMEM((NW, 3, L), jnp.float32),
        pltpu.VMEM((2 * L,), jnp.float32),
        pltpu.SemaphoreType.DMA, pltpu.SemaphoreType.DMA,
        pltpu.SemaphoreType.DMA, pltpu.SemaphoreType.DMA,
        pltpu.SemaphoreType.DMA, pltpu.SemaphoreType.DMA,
        pltpu.SemaphoreType.DMA, pltpu.SemaphoreType.DMA,
        pltpu.SemaphoreType.DMA,
        pltpu.SemaphoreType.REGULAR,
    ],
)
def _fused(x_hbm, c_hbm, i_hbm, out_hbm, parts_hbm,
           xb0, xb1, cb0, cb1, ib0, ib1, ob0, ob1, pme, pall, rbuf,
           sx0, sx1, sc0, sc1, si0, si1, so0, so1, spub, xsem):
    cid = lax.axis_index("c")
    w = lax.axis_index("s") * 2 + cid
    bstart, nblk = _tile_range(w)
    zero = jnp.zeros((L,), jnp.float32)

    in_slots = ((xb0, cb0, ib0, sx0, sc0, si0),
                (xb1, cb1, ib1, sx1, sc1, si1))

    def issue1(b, s):
        xb, cb, ib, sx, sc, si = in_slots[s]
        base = pl.multiple_of((bstart + b) * BLK, BLK)
        pltpu.async_copy(x_hbm.at[pl.ds(base, BLK)], xb, sx)
        pltpu.async_copy(c_hbm.at[pl.ds(base, BLK)], cb, sc)
        pltpu.async_copy(i_hbm.at[pl.ds(base, BLK)], ib, si)

    def wait1(s):
        xb, cb, ib, sx, sc, si = in_slots[s]
        src = x_hbm.at[pl.ds(0, BLK)]
        pltpu.make_async_copy(src, xb, sx).wait()
        pltpu.make_async_copy(src, cb, sc).wait()
        isrc = i_hbm.at[pl.ds(0, BLK)]
        pltpu.make_async_copy(isrc, ib, si).wait()

    def acc_block(s):
        xb, cb, ib = in_slots[s][:3]
        carry = (pme[0, :], pme[1, :], pme[2, :], zero, zero, zero)

        def vec_body(i, carry2):
            tot_a, s1_a, cnt_a, tot_b, s1_b, cnt_b = carry2
            xa = xb[pl.ds(i, L)]
            ca = cb[pl.ds(i, L)]
            ia = ib[pl.ds(i, L)]
            pa = xa * ca
            fa = ia.astype(jnp.float32)
            xbv = xb[pl.ds(i + L, L)]
            cbv = cb[pl.ds(i + L, L)]
            ibv = ib[pl.ds(i + L, L)]
            pb = xbv * cbv
            fb = ibv.astype(jnp.float32)
            return (tot_a + pa, s1_a + pa * fa, cnt_a + fa,
                    tot_b + pb, s1_b + pb * fb, cnt_b + fb)

        r = plsc.parallel_loop(0, BLK, 2 * L, unroll=2,
                               carry=carry)(vec_body)
        pme[0, :] = r[0] + r[3]
        pme[1, :] = r[1] + r[4]
        pme[2, :] = r[2] + r[5]

    pme[0, :] = zero
    pme[1, :] = zero
    pme[2, :] = zero
    issue1(0, 0)
    issue1(1, 1)

    def pair1(pr, _):
        b = 2 * pr
        wait1(0)
        acc_block(0)

        @pl.when(b + 2 < nblk)
        def _():
            issue1(b + 2, 0)

        wait1(1)
        acc_block(1)

        @pl.when(b + 3 < nblk)
        def _():
            issue1(b + 3, 1)

        return 0

    lax.fori_loop(0, nblk // 2, pair1, 0)

    @pl.when(lax.rem(nblk, 2) == 1)
    def _():
        wait1(0)
        acc_block(0)

    pltpu.async_copy(pme, parts_hbm.at[w], spub).wait()

    out_slots = ((xb0, cb0, ob0, sx0, sc0, so0),
                 (xb1, cb1, ob1, sx1, sc1, so1))

    def issue2(b, s):
        xb, cb = out_slots[s][:2]
        sx, sc = out_slots[s][3:5]
        base = pl.multiple_of((bstart + b) * BLK, BLK)
        pltpu.async_copy(x_hbm.at[pl.ds(base, BLK)], xb, sx)
        pltpu.async_copy(c_hbm.at[pl.ds(base, BLK)], cb, sc)

    issue2(0, 0)
    issue2(1, 1)

    plsc.subcore_barrier()
    pl.semaphore_signal(xsem, 1, core_index=1 - cid)
    pl.semaphore_wait(xsem, 1)

    pltpu.sync_copy(parts_hbm, pall)

    def all_lanes_sum(v):
        for k in (8, 4, 2, 1):
            rbuf[pl.ds(0, L)] = v
            rbuf[pl.ds(L, L)] = v
            v = v + rbuf[pl.ds(k, L)]
        return v

    totv = jnp.zeros((L,), jnp.float32)
    s1v = jnp.zeros((L,), jnp.float32)
    cntv = jnp.zeros((L,), jnp.float32)
    for t in range(NW):
        totv = totv + pall[t, 0, :]
        s1v = s1v + pall[t, 1, :]
        cntv = cntv + pall[t, 2, :]
    s1_all = all_lanes_sum(s1v)
    s0 = (all_lanes_sum(totv) - s1_all) * INV_N_CHARGES
    s1 = s1_all * INV_N_CHARGES
    bndf = jnp.float32(N) - all_lanes_sum(cntv)
    bnd = bndf.astype(jnp.int32)
    bnd_s = bndf[0].astype(jnp.int32)
    iota = lax.iota(jnp.int32, L)

    def wait2(s):
        xb, cb = out_slots[s][:2]
        sx, sc = out_slots[s][3:5]
        src = x_hbm.at[pl.ds(0, BLK)]
        pltpu.make_async_copy(src, xb, sx).wait()
        pltpu.make_async_copy(src, cb, sc).wait()

    def wait_scatter(s):
        ob, so = out_slots[s][2], out_slots[s][5]
        pltpu.make_async_copy(ob, out_hbm.at[pl.ds(0, BLK)], so).wait()

    def process(b, s):
        xb, cb, ob = out_slots[s][:3]
        so = out_slots[s][5]
        base = pl.multiple_of((bstart + b) * BLK, BLK)

        def const_body(segv):
            def vec_body(i):
                xv = xb[pl.ds(i, L)]
                cv = cb[pl.ds(i, L)]
                ob[pl.ds(i, L)] = (xv - segv) * cv
            return vec_body

        def sel_body(i):
            xv = xb[pl.ds(i, L)]
            cv = cb[pl.ds(i, L)]
            pos = iota + (base + i)
            sel = jnp.where(pos < bnd, s0, s1)
            ob[pl.ds(i, L)] = (xv - sel) * cv

        below = base + BLK <= bnd_s
        above = base >= bnd_s

        @pl.when(below)
        def _():
            plsc.parallel_loop(0, BLK, L, unroll=4)(const_body(s0))

        @pl.when(above)
        def _():
            plsc.parallel_loop(0, BLK, L, unroll=4)(const_body(s1))

        @pl.when(jnp.logical_not(jnp.logical_or(below, above)))
        def _():
            plsc.parallel_loop(0, BLK, L, unroll=4)(sel_body)

        pltpu.async_copy(ob, out_hbm.at[pl.ds(base, BLK)], so)

    def pair2(pr, _):
        b = 2 * pr
        wait2(0)

        @pl.when(pr > 0)
        def _():
            wait_scatter(0)

        process(b, 0)

        @pl.when(b + 2 < nblk)
        def _():
            issue2(b + 2, 0)

        wait2(1)

        @pl.when(pr > 0)
        def _():
            wait_scatter(1)

        process(b + 1, 1)

        @pl.when(b + 3 < nblk)
        def _():
            issue2(b + 3, 1)

        return 0

    lax.fori_loop(0, nblk // 2, pair2, 0)

    @pl.when(lax.rem(nblk, 2) == 1)
    def _():
        wait2(0)
        wait_scatter(0)
        process(nblk - 1, 0)

    wait_scatter(0)
    wait_scatter(1)


def kernel(x0, charges, chg_idx):
    idx32 = chg_idx.astype(jnp.int32)
    out, _ = _fused(x0, charges, idx32)
    return out
